# baseline (speedup 1.0000x reference)
import functools

import jax
import jax.numpy as jnp
from jax import lax
from jax.experimental import pallas as pl
from jax.experimental.pallas import tpu as pltpu
from jax.experimental.pallas import tpu_sc as plsc

_B = 4
_N = 8192
_D = 1024

_L = 16
_CHUNK = _N // _L
_NBINS = 256
_NPASS = 4

_SIGN = -(2**31)


def _digit(k, shift):
    return jnp.bitwise_and(k >> shift, _NBINS - 1)


def _sc_sort_body(met_hbm, sidx_hbm, metv, keys0, vals0, keys1, vals1,
                  hist, excl, tot, gx, run, sidx_v):
    c = lax.axis_index("c")
    s = lax.axis_index("s")
    wid = s * 2 + c

    @pl.when(wid < _B)
    def _():
        b = wid
        lane = lax.iota(jnp.int32, _L)
        ones = jnp.ones((_L,), jnp.int32)

        pltpu.sync_copy(met_hbm.at[b], metv)

        def init_t(t, cr):
            f = metv[pl.ds(t * _L, _L)]
            bits = lax.bitcast_convert_type(f, jnp.int32)
            bits = jnp.where(f == 0.0, jnp.int32(0), bits)
            m = bits >> 31
            k = lax.bitwise_xor(bits, lax.bitwise_or(m, jnp.full((_L,), _SIGN, jnp.int32)))
            keys0[pl.ds(t * _L, _L)] = k
            vals0[pl.ds(t * _L, _L)] = t * _L + lane
            return cr

        lax.fori_loop(0, _N // _L, init_t, 0)

        for p in range(_NPASS):
            shift = 8 * p
            last = p == _NPASS - 1
            keys_in = keys0 if p % 2 == 0 else keys1
            vals_in = vals0 if p % 2 == 0 else vals1
            keys_out = keys1 if p % 2 == 0 else keys0
            vals_out = vals1 if p % 2 == 0 else vals0

            def zero_h(v4, cr):
                for u in range(4):
                    hist[pl.ds((v4 * 4 + u) * _L, _L)] = jnp.zeros(
                        (_L,), jnp.int32)
                return cr

            lax.fori_loop(0, _NBINS // 4, zero_h, 0)

            def count_t(t4, cr, keys_in=keys_in, shift=shift):
                for u in range(4):
                    idx = lane * _CHUNK + (t4 * 4 + u)
                    k = plsc.load_gather(keys_in, [idx])
                    d = _digit(k, shift)
                    plsc.addupdate_scatter(hist, [d * _L + lane], ones)
                return cr

            lax.fori_loop(0, _CHUNK // 4, count_t, 0)

            def scan_d(v4, cr):
                for u in range(4):
                    v = v4 * 4 + u
                    row = hist[pl.ds(v * _L, _L)]
                    incl = plsc.cumsum(row)
                    excl[pl.ds(v * _L, _L)] = incl - row
                    tv = jnp.sum(row)
                    plsc.store_scatter(tot, [jnp.full((_L,), v, jnp.int32)],
                                       jnp.full((_L,), tv, jnp.int32),
                                       mask=lane == 0)
                return cr

            lax.fori_loop(0, _NBINS // 4, scan_d, 0)

            def scan_g(w, carry):
                tw = tot[pl.ds(w * _L, _L)]
                incl = plsc.cumsum(tw) + carry
                gx[pl.ds(w * _L, _L)] = incl - tw
                return carry + jnp.sum(tw)

            lax.fori_loop(0, _NBINS // _L, scan_g, jnp.int32(0))

            def init_run(v4, cr):
                for u in range(4):
                    v = v4 * 4 + u
                    g = plsc.load_gather(gx, [jnp.full((_L,), v, jnp.int32)])
                    run[pl.ds(v * _L, _L)] = g + excl[pl.ds(v * _L, _L)]
                return cr

            lax.fori_loop(0, _NBINS // 4, init_run, 0)

            def perm_t(t4, cr, keys_in=keys_in, vals_in=vals_in,
                       keys_out=keys_out, vals_out=vals_out,
                       shift=shift, last=last):
                for u in range(4):
                    idx = lane * _CHUNK + (t4 * 4 + u)
                    k = plsc.load_gather(keys_in, [idx])
                    val = plsc.load_gather(vals_in, [idx])
                    d = _digit(k, shift)
                    hidx = d * _L + lane
                    pos = plsc.load_gather(run, [hidx])
                    plsc.store_scatter(run, [hidx], pos + 1)
                    if last:
                        plsc.store_scatter(sidx_v, [val], pos + b * _N)
                    else:
                        plsc.store_scatter(keys_out, [pos], k)
                        plsc.store_scatter(vals_out, [pos], val)
                return cr

            lax.fori_loop(0, _CHUNK // 4, perm_t, 0)

        pltpu.sync_copy(sidx_v, sidx_hbm.at[pl.ds(b * _N, _N)])


@functools.cache
def _sc_sort():
    return pl.kernel(
        _sc_sort_body,
        out_type=jax.ShapeDtypeStruct((_B * _N,), jnp.int32),
        mesh=plsc.VectorSubcoreMesh(core_axis_name="c", subcore_axis_name="s"),
        compiler_params=pltpu.CompilerParams(needs_layout_passes=False),
        scratch_types=[
            pltpu.VMEM((_N,), jnp.float32),
            pltpu.VMEM((_N,), jnp.int32),
            pltpu.VMEM((_N,), jnp.int32),
            pltpu.VMEM((_N,), jnp.int32),
            pltpu.VMEM((_N,), jnp.int32),
            pltpu.VMEM((_NBINS * _L,), jnp.int32),
            pltpu.VMEM((_NBINS * _L,), jnp.int32),
            pltpu.VMEM((_NBINS,), jnp.int32),
            pltpu.VMEM((_NBINS,), jnp.int32),
            pltpu.VMEM((_NBINS * _L,), jnp.int32),
            pltpu.VMEM((_N,), jnp.int32),
        ],
    )


_NW = 32
_RPW = (_B * _N) // _NW
_CH = 32
_NCH = _RPW // _CH


def _sc_scatter_body(x_hbm, sidx_hbm, out_hbm, idx0, idx1, xb0, xb1,
                     is0, is1, rs0, rs1, ws0, ws1):
    c = lax.axis_index("c")
    s = lax.axis_index("s")
    wid = s * 2 + c
    base = wid * _RPW
    bufs = ((idx0, xb0, is0, rs0, ws0), (idx1, xb1, is1, rs1, ws1))

    def start_read(g, idx_b, x_b, isem, rsem):
        st = base + g * _CH
        pltpu.make_async_copy(sidx_hbm.at[pl.ds(st, _CH)], idx_b, isem).start()
        pltpu.make_async_copy(x_hbm.at[pl.ds(st, _CH)], x_b, rsem).start()

    def wait_read(idx_b, x_b, isem, rsem):
        pltpu.make_async_copy(sidx_hbm.at[pl.ds(base, _CH)], idx_b, isem).wait()
        pltpu.make_async_copy(x_hbm.at[pl.ds(base, _CH)], x_b, rsem).wait()

    def write(idx_b, x_b, wsem):
        wcopy = pltpu.make_async_copy(x_b, out_hbm.at[idx_b], wsem)
        wcopy.start()
        wcopy.wait()

    start_read(0, idx0, xb0, is0, rs0)
    start_read(1, idx1, xb1, is1, rs1)

    def pair(h, cr):
        g0 = h * 2
        for par, (idx_b, x_b, isem, rsem, wsem) in enumerate(bufs):
            wait_read(idx_b, x_b, isem, rsem)
            write(idx_b, x_b, wsem)
            start_read(g0 + par + 2, idx_b, x_b, isem, rsem)
        return cr

    lax.fori_loop(0, _NCH // 2 - 1, pair, 0)

    for idx_b, x_b, isem, rsem, wsem in bufs:
        wait_read(idx_b, x_b, isem, rsem)
        write(idx_b, x_b, wsem)


@functools.cache
def _sc_scatter():
    return pl.kernel(
        _sc_scatter_body,
        out_type=jax.ShapeDtypeStruct((_B * _N, _D), jnp.float32),
        mesh=plsc.VectorSubcoreMesh(core_axis_name="c", subcore_axis_name="s"),
        scratch_types=[
            pltpu.VMEM((_CH,), jnp.int32),
            pltpu.VMEM((_CH,), jnp.int32),
            pltpu.VMEM((_CH, _D), jnp.float32),
            pltpu.VMEM((_CH, _D), jnp.float32),
            pltpu.SemaphoreType.DMA,
            pltpu.SemaphoreType.DMA,
            pltpu.SemaphoreType.DMA,
            pltpu.SemaphoreType.DMA,
            pltpu.SemaphoreType.DMA,
            pltpu.SemaphoreType.DMA,
        ],
    )


def kernel(met, X):
    sidx = _sc_sort()(met)
    out = _sc_scatter()(X.reshape(_B * _N, _D), sidx)
    return out.reshape(_B, _N, _D)


# speedup vs baseline: 1.0372x; 1.0372x over previous
import functools

import jax
import jax.numpy as jnp
from jax import lax
from jax.experimental import pallas as pl
from jax.experimental.pallas import tpu as pltpu
from jax.experimental.pallas import tpu_sc as plsc

_B = 4
_N = 8192
_D = 1024

_L = 16
_CHUNK = _N // _L
_NBINS = 256
_NPASS = 4

_SIGN = -(2**31)

_RPW = 1024
_CH = 32
_NCH = _RPW // _CH


def _digit(k, shift):
    return jnp.bitwise_and(k >> shift, _NBINS - 1)


def _fused_body(met_hbm, x_hbm, out_hbm, sidx_hbm,
                metv, keys0, vals0, keys1, vals1,
                hist, excl, tot, gx, run, sidx_v,
                idx0, idx1, xb0, xb1, is0, is1, rs0, rs1, ws0, ws1):
    c = lax.axis_index("c")
    s = lax.axis_index("s")
    base = c * (_B * _N // 2) + s * _RPW

    def start_xread(g, x_b, rsem):
        pltpu.make_async_copy(
            x_hbm.at[pl.ds(base + g * _CH, _CH)], x_b, rsem).start()

    def start_iread(g, idx_b, isem):
        pltpu.make_async_copy(
            sidx_hbm.at[pl.ds(base + g * _CH, _CH)], idx_b, isem).start()

    def wait_read(idx_b, x_b, isem, rsem):
        pltpu.make_async_copy(sidx_hbm.at[pl.ds(base, _CH)], idx_b, isem).wait()
        pltpu.make_async_copy(x_hbm.at[pl.ds(base, _CH)], x_b, rsem).wait()

    def write(idx_b, x_b, wsem):
        wcopy = pltpu.make_async_copy(x_b, out_hbm.at[idx_b], wsem)
        wcopy.start()
        wcopy.wait()

    start_xread(0, xb0, rs0)
    start_xread(1, xb1, rs1)

    @pl.when(s < 2)
    def _():
        b = c * 2 + s
        lane = lax.iota(jnp.int32, _L)
        ones = jnp.ones((_L,), jnp.int32)

        pltpu.sync_copy(met_hbm.at[b], metv)

        def init_t(t, cr):
            f = metv[pl.ds(t * _L, _L)]
            bits = lax.bitcast_convert_type(f, jnp.int32)
            bits = jnp.where(f == 0.0, jnp.int32(0), bits)
            m = bits >> 31
            k = lax.bitwise_xor(
                bits, lax.bitwise_or(m, jnp.full((_L,), _SIGN, jnp.int32)))
            keys0[pl.ds(t * _L, _L)] = k
            vals0[pl.ds(t * _L, _L)] = t * _L + lane
            return cr

        lax.fori_loop(0, _N // _L, init_t, 0)

        for p in range(_NPASS):
            shift = 8 * p
            last = p == _NPASS - 1
            keys_in = keys0 if p % 2 == 0 else keys1
            vals_in = vals0 if p % 2 == 0 else vals1
            keys_out = keys1 if p % 2 == 0 else keys0
            vals_out = vals1 if p % 2 == 0 else vals0

            def zero_h(v, cr):
                hist[pl.ds(v * _L, _L)] = jnp.zeros((_L,), jnp.int32)
                return cr

            lax.fori_loop(0, _NBINS, zero_h, 0)

            def count_t(t, cr, keys_in=keys_in, shift=shift):
                idx = lane * _CHUNK + t
                k = plsc.load_gather(keys_in, [idx])
                d = _digit(k, shift)
                plsc.addupdate_scatter(hist, [d * _L + lane], ones)
                return cr

            lax.fori_loop(0, _CHUNK, count_t, 0)

            def scan_d(v, cr):
                row = hist[pl.ds(v * _L, _L)]
                incl = plsc.cumsum(row)
                excl[pl.ds(v * _L, _L)] = incl - row
                tv = jnp.sum(row)
                plsc.store_scatter(tot, [jnp.full((_L,), v, jnp.int32)],
                                   jnp.full((_L,), tv, jnp.int32),
                                   mask=lane == 0)
                return cr

            lax.fori_loop(0, _NBINS, scan_d, 0)

            def scan_g(w, carry):
                tw = tot[pl.ds(w * _L, _L)]
                incl = plsc.cumsum(tw) + carry
                gx[pl.ds(w * _L, _L)] = incl - tw
                return carry + jnp.sum(tw)

            lax.fori_loop(0, _NBINS // _L, scan_g, jnp.int32(0))

            def init_run(v, cr):
                g = plsc.load_gather(gx, [jnp.full((_L,), v, jnp.int32)])
                run[pl.ds(v * _L, _L)] = g + excl[pl.ds(v * _L, _L)]
                return cr

            lax.fori_loop(0, _NBINS, init_run, 0)

            def perm_t(t, cr, keys_in=keys_in, vals_in=vals_in,
                       keys_out=keys_out, vals_out=vals_out,
                       shift=shift, last=last):
                idx = lane * _CHUNK + t
                k = plsc.load_gather(keys_in, [idx])
                val = plsc.load_gather(vals_in, [idx])
                d = _digit(k, shift)
                hidx = d * _L + lane
                pos = plsc.load_gather(run, [hidx])
                plsc.store_scatter(run, [hidx], pos + 1)
                if last:
                    plsc.store_scatter(sidx_v, [val], pos + b * _N)
                else:
                    plsc.store_scatter(keys_out, [pos], k)
                    plsc.store_scatter(vals_out, [pos], val)
                return cr

            lax.fori_loop(0, _CHUNK, perm_t, 0)

        pltpu.sync_copy(sidx_v, sidx_hbm.at[pl.ds(b * _N, _N)])

    plsc.subcore_barrier()

    start_iread(0, idx0, is0)
    start_iread(1, idx1, is1)

    bufs = ((idx0, xb0, is0, rs0, ws0), (idx1, xb1, is1, rs1, ws1))

    def pair(h, cr):
        g0 = h * 2
        for par, (idx_b, x_b, isem, rsem, wsem) in enumerate(bufs):
            wait_read(idx_b, x_b, isem, rsem)
            write(idx_b, x_b, wsem)
            start_xread(g0 + par + 2, x_b, rsem)
            start_iread(g0 + par + 2, idx_b, isem)
        return cr

    lax.fori_loop(0, _NCH // 2 - 1, pair, 0)

    for idx_b, x_b, isem, rsem, wsem in bufs:
        wait_read(idx_b, x_b, isem, rsem)
        write(idx_b, x_b, wsem)


@functools.cache
def _fused():
    return pl.kernel(
        _fused_body,
        out_type=(
            jax.ShapeDtypeStruct((_B * _N, _D), jnp.float32),
            jax.ShapeDtypeStruct((_B * _N,), jnp.int32),
        ),
        mesh=plsc.VectorSubcoreMesh(core_axis_name="c", subcore_axis_name="s"),
        compiler_params=pltpu.CompilerParams(needs_layout_passes=False),
        scratch_types=[
            pltpu.VMEM((_N,), jnp.float32),
            pltpu.VMEM((_N,), jnp.int32),
            pltpu.VMEM((_N,), jnp.int32),
            pltpu.VMEM((_N,), jnp.int32),
            pltpu.VMEM((_N,), jnp.int32),
            pltpu.VMEM((_NBINS * _L,), jnp.int32),
            pltpu.VMEM((_NBINS * _L,), jnp.int32),
            pltpu.VMEM((_NBINS,), jnp.int32),
            pltpu.VMEM((_NBINS,), jnp.int32),
            pltpu.VMEM((_NBINS * _L,), jnp.int32),
            pltpu.VMEM((_N,), jnp.int32),
            pltpu.VMEM((_CH,), jnp.int32),
            pltpu.VMEM((_CH,), jnp.int32),
            pltpu.VMEM((_CH, _D), jnp.float32),
            pltpu.VMEM((_CH, _D), jnp.float32),
            pltpu.SemaphoreType.DMA,
            pltpu.SemaphoreType.DMA,
            pltpu.SemaphoreType.DMA,
            pltpu.SemaphoreType.DMA,
            pltpu.SemaphoreType.DMA,
            pltpu.SemaphoreType.DMA,
        ],
    )


def kernel(met, X):
    out, _ = _fused()(met, X.reshape(_B * _N, _D))
    return out.reshape(_B, _N, _D)
